# baseline (speedup 1.0000x reference)
import jax
import jax.numpy as jnp
from jax import lax
from jax.experimental import pallas as pl

_N = 4096
_KB = 256


def _tc_body(basis_ref, bsq_ref, pct_ref, pcsq_ref, pc_ref, out_ref):
    ba = basis_ref[...]
    pct = pct_ref[0]
    cross = jnp.dot(ba, pct, preferred_element_type=jnp.float32)
    sq = (bsq_ref[...] + pcsq_ref[0]) - 2.0 * cross
    m = jnp.min(sq, axis=1, keepdims=True)
    iota = lax.broadcasted_iota(jnp.int32, sq.shape, 1)
    idx = jnp.min(jnp.where(sq == m, iota, _N), axis=1, keepdims=True)
    onehot = (iota == idx).astype(jnp.float32)
    pc = pc_ref[0]
    near = jnp.dot(onehot, pc, preferred_element_type=jnp.float32,
                   precision=lax.Precision.HIGHEST)
    deltas = near[:, 0:3] - ba[:, 0:3]
    dists = jnp.sqrt(jnp.sum(deltas * deltas, axis=1, keepdims=True) + 1e-12)
    out_ref[0] = jnp.concatenate([dists, deltas], axis=1)


def kernel(point_cloud, basis):
    B, N, _ = point_cloud.shape
    K = basis.shape[0]
    pc_sq = jnp.sum(point_cloud * point_cloud, axis=-1)
    b_sq = jnp.sum(basis * basis, axis=-1, keepdims=True)

    zeros5 = jnp.zeros((B, N, 5), jnp.float32)
    pc8 = jnp.concatenate([point_cloud, zeros5], axis=-1)
    pc8_t = jnp.transpose(pc8, (0, 2, 1))
    basis8 = jnp.concatenate(
        [basis, jnp.zeros((K, 5), jnp.float32)], axis=-1)
    pc_sq3 = pc_sq[:, None, :]

    grid = (B, K // _KB)
    out = pl.pallas_call(
        _tc_body,
        grid=grid,
        in_specs=[
            pl.BlockSpec((_KB, 8), lambda b, kg: (kg, 0)),
            pl.BlockSpec((_KB, 1), lambda b, kg: (kg, 0)),
            pl.BlockSpec((1, 8, N), lambda b, kg: (b, 0, 0)),
            pl.BlockSpec((1, 1, N), lambda b, kg: (b, 0, 0)),
            pl.BlockSpec((1, N, 8), lambda b, kg: (b, 0, 0)),
        ],
        out_specs=pl.BlockSpec((1, _KB, 4), lambda b, kg: (b, kg, 0)),
        out_shape=jax.ShapeDtypeStruct((B, K, 4), jnp.float32),
    )(basis8, b_sq, pc8_t, pc_sq3, pc8)
    return out


# speedup vs baseline: 3.5471x; 3.5471x over previous
import functools

import jax
import jax.numpy as jnp
from jax import lax
from jax.experimental import pallas as pl
from jax.experimental.pallas import tpu as pltpu
from jax.experimental.pallas import tpu_sc as plsc

_B = 8
_N = 4096
_K = 4096
_KB = 256

_NC = 2
_NS = 16
_NW = _NC * _NS
_CHUNK = (_B * _K) // _NW


def _tc_body(basis_ref, bsq_ref, pct_ref, pcsq_ref, idx_ref):
    ba = basis_ref[...]
    pct = pct_ref[0]
    cross = jnp.dot(ba, pct, preferred_element_type=jnp.float32)
    sq = (bsq_ref[...] + pcsq_ref[0]) - 2.0 * cross
    m = jnp.min(sq, axis=1, keepdims=True)
    iota = lax.broadcasted_iota(jnp.int32, sq.shape, 1)
    idx = jnp.min(jnp.where(sq == m, iota, _N), axis=1, keepdims=True)
    idx_ref[0] = idx.reshape(1, _KB)


def _nn_indices(basis8, b_sq, pc8_t, pc_sq3):
    kg = _K // _KB
    idx = pl.pallas_call(
        _tc_body,
        grid=(_B, kg),
        in_specs=[
            pl.BlockSpec((_KB, 8), lambda b, g: (g, 0)),
            pl.BlockSpec((_KB, 1), lambda b, g: (g, 0)),
            pl.BlockSpec((1, 8, _N), lambda b, g: (b, 0, 0)),
            pl.BlockSpec((1, 1, _N), lambda b, g: (b, 0, 0)),
        ],
        out_specs=pl.BlockSpec((1, 1, _KB), lambda b, g: (b * kg + g, 0, 0)),
        out_shape=jax.ShapeDtypeStruct((_B * kg, 1, _KB), jnp.int32),
    )(basis8, b_sq, pc8_t, pc_sq3)
    return idx.reshape(_B * _K)


def _rsqrt_newton(ss):
    bits = plsc.bitcast(ss, jnp.int32)
    y = plsc.bitcast(0x5F3759DF - lax.shift_right_arithmetic(bits, 1),
                     jnp.float32)
    for _ in range(3):
        y = y * (1.5 - 0.5 * ss * y * y)
    return y


def _sc_gather(pcx, pcy, pcz, bx, by, bz, idx):
    mesh = plsc.VectorSubcoreMesh(core_axis_name="c", subcore_axis_name="s")
    fdt = jax.ShapeDtypeStruct((_B * _K,), jnp.float32)

    @functools.partial(
        pl.kernel,
        out_type=(fdt, fdt, fdt, fdt),
        mesh=mesh,
        compiler_params=pltpu.CompilerParams(needs_layout_passes=False),
        scratch_types=[
            pltpu.VMEM((_CHUNK,), jnp.int32),
            pltpu.VMEM((_N,), jnp.float32),
            pltpu.VMEM((_N,), jnp.float32),
            pltpu.VMEM((_N,), jnp.float32),
            pltpu.VMEM((_CHUNK,), jnp.float32),
            pltpu.VMEM((_CHUNK,), jnp.float32),
            pltpu.VMEM((_CHUNK,), jnp.float32),
            pltpu.VMEM((_CHUNK,), jnp.float32),
            pltpu.VMEM((_CHUNK,), jnp.float32),
            pltpu.VMEM((_CHUNK,), jnp.float32),
            pltpu.VMEM((_CHUNK,), jnp.float32),
        ],
    )
    def run(pcx_h, pcy_h, pcz_h, bx_h, by_h, bz_h, idx_h,
            od_h, ox_h, oy_h, oz_h,
            idx_v, px_v, py_v, pz_v, bx_v, by_v, bz_v,
            od_v, ox_v, oy_v, oz_v):
        wid = lax.axis_index("s") * _NC + lax.axis_index("c")
        base = wid * _CHUNK
        b = base // _K
        koff = base - b * _K
        pltpu.sync_copy(pcx_h.at[pl.ds(b * _N, _N)], px_v)
        pltpu.sync_copy(pcy_h.at[pl.ds(b * _N, _N)], py_v)
        pltpu.sync_copy(pcz_h.at[pl.ds(b * _N, _N)], pz_v)
        pltpu.sync_copy(bx_h.at[pl.ds(koff, _CHUNK)], bx_v)
        pltpu.sync_copy(by_h.at[pl.ds(koff, _CHUNK)], by_v)
        pltpu.sync_copy(bz_h.at[pl.ds(koff, _CHUNK)], bz_v)
        pltpu.sync_copy(idx_h.at[pl.ds(base, _CHUNK)], idx_v)

        def body(i, carry):
            off = i * 16
            iv = idx_v[pl.ds(off, 16)]
            nx = plsc.load_gather(px_v, [iv])
            ny = plsc.load_gather(py_v, [iv])
            nz = plsc.load_gather(pz_v, [iv])
            dx = nx - bx_v[pl.ds(off, 16)]
            dy = ny - by_v[pl.ds(off, 16)]
            dz = nz - bz_v[pl.ds(off, 16)]
            ss = dx * dx + dy * dy + dz * dz + 1e-12
            d = ss * _rsqrt_newton(ss)
            od_v[pl.ds(off, 16)] = d
            ox_v[pl.ds(off, 16)] = dx
            oy_v[pl.ds(off, 16)] = dy
            oz_v[pl.ds(off, 16)] = dz
            return carry

        lax.fori_loop(0, _CHUNK // 16, body, 0)

        pltpu.sync_copy(od_v, od_h.at[pl.ds(base, _CHUNK)])
        pltpu.sync_copy(ox_v, ox_h.at[pl.ds(base, _CHUNK)])
        pltpu.sync_copy(oy_v, oy_h.at[pl.ds(base, _CHUNK)])
        pltpu.sync_copy(oz_v, oz_h.at[pl.ds(base, _CHUNK)])

    return run(pcx, pcy, pcz, bx, by, bz, idx)


def kernel(point_cloud, basis):
    B, N, _ = point_cloud.shape
    K = basis.shape[0]
    pc_sq = jnp.sum(point_cloud * point_cloud, axis=-1)
    b_sq = jnp.sum(basis * basis, axis=-1, keepdims=True)

    pc8_t = jnp.concatenate(
        [jnp.transpose(point_cloud, (0, 2, 1)),
         jnp.zeros((B, 5, N), jnp.float32)], axis=1)
    basis8 = jnp.concatenate(
        [basis, jnp.zeros((K, 5), jnp.float32)], axis=-1)
    pc_sq3 = pc_sq[:, None, :]

    idx = _nn_indices(basis8, b_sq, pc8_t, pc_sq3)

    pcx = point_cloud[:, :, 0].reshape(B * N)
    pcy = point_cloud[:, :, 1].reshape(B * N)
    pcz = point_cloud[:, :, 2].reshape(B * N)
    d, dx, dy, dz = _sc_gather(pcx, pcy, pcz,
                               basis[:, 0], basis[:, 1], basis[:, 2], idx)
    out = jnp.stack([d, dx, dy, dz], axis=-1)
    return out.reshape(B, K, 4)
